# baseline (speedup 1.0000x reference)
import functools

import jax
import jax.numpy as jnp
from jax import lax
from jax.experimental import pallas as pl
from jax.experimental.pallas import tpu as pltpu
from jax.experimental.pallas import tpu_sc as plsc

_VB = 2048
_IDX_CHUNK = 128


def _gather_mean_sc(idx_flat, emb, batch, win):
    info = plsc.get_sparse_core_info()
    nc, ns = info.num_cores, info.num_subcores
    nw = nc * ns
    edim = emb.shape[1]
    b_per_w = batch // nw
    rows_per_w = b_per_w * win
    n_chunks = rows_per_w // _IDX_CHUNK
    mesh = plsc.VectorSubcoreMesh(core_axis_name="c", subcore_axis_name="s")

    def body(idx_hbm, emb_hbm, out_hbm, idx_v, rows_v, acc_v, sem):
        wid = lax.axis_index("s") * nc + lax.axis_index("c")
        base = wid * rows_per_w
        pltpu.sync_copy(idx_hbm.at[pl.ds(base, rows_per_w)], idx_v)
        copies = []
        for j in range(n_chunks):
            copies.append(pltpu.async_copy(
                emb_hbm.at[idx_v.at[pl.ds(j * _IDX_CHUNK, _IDX_CHUNK)]],
                rows_v.at[pl.ds(j * _IDX_CHUNK, _IDX_CHUNK)],
                sem))
        for c in copies:
            c.wait()

        def pool_one(i, carry):
            acc = rows_v[i * win, :]
            for j in range(1, win):
                acc = acc + rows_v[i * win + j, :]
            acc_v[i, :] = acc * (1.0 / win)
            return carry

        lax.fori_loop(0, b_per_w, pool_one, 0)
        pltpu.sync_copy(acc_v, out_hbm.at[pl.ds(wid * b_per_w, b_per_w)])

    kfn = pl.kernel(
        body,
        mesh=mesh,
        compiler_params=pltpu.CompilerParams(use_tc_tiling_on_sc=False),
        out_type=jax.ShapeDtypeStruct((batch, edim), jnp.float32),
        scratch_types=[
            pltpu.VMEM((rows_per_w,), jnp.int32),
            pltpu.VMEM((rows_per_w, edim), jnp.float32),
            pltpu.VMEM((b_per_w, edim), jnp.float32),
            pltpu.SemaphoreType.DMA,
        ],
    )
    return kfn(idx_flat, emb)


def _lse_tc(vb, wbt, vocab):
    batch, k = vb.shape
    nv = pl.cdiv(vocab, _VB)

    def body(vb_ref, wbt_ref, lse_ref, m_s, s_s):
        iv = pl.program_id(0)
        logits = jnp.dot(vb_ref[:], wbt_ref[:],
                         preferred_element_type=jnp.float32)
        col = iv * _VB + lax.broadcasted_iota(jnp.int32, logits.shape, 1)
        logits = jnp.where(col < vocab, logits, -1e30)
        tmax = jnp.max(logits, axis=1, keepdims=True)

        @pl.when(iv == 0)
        def _():
            m_s[:] = tmax
            s_s[:] = jnp.sum(jnp.exp(logits - tmax), axis=1, keepdims=True)

        @pl.when(iv != 0)
        def _():
            m_old = m_s[:]
            m_new = jnp.maximum(m_old, tmax)
            s_s[:] = (s_s[:] * jnp.exp(m_old - m_new)
                      + jnp.sum(jnp.exp(logits - m_new), axis=1,
                                keepdims=True))
            m_s[:] = m_new

        @pl.when(iv == nv - 1)
        def _():
            lse_ref[:] = m_s[:] + jnp.log(s_s[:])

    return pl.pallas_call(
        body,
        grid=(nv,),
        in_specs=[
            pl.BlockSpec((batch, k), lambda i: (0, 0)),
            pl.BlockSpec((k, _VB), lambda i: (0, i)),
        ],
        out_specs=pl.BlockSpec((batch, 1), lambda i: (0, 0)),
        out_shape=jax.ShapeDtypeStruct((batch, 1), jnp.float32),
        scratch_shapes=[
            pltpu.VMEM((batch, 1), jnp.float32),
            pltpu.VMEM((batch, 1), jnp.float32),
        ],
    )(vb, wbt)


def _logsoftmax_out_tc(vb, wbt, lse, vocab):
    batch, k = vb.shape
    nv = pl.cdiv(vocab, _VB)

    def body(vb_ref, wbt_ref, lse_ref, o_ref):
        logits = jnp.dot(vb_ref[:], wbt_ref[:],
                         preferred_element_type=jnp.float32)
        o_ref[:] = logits - lse_ref[:]

    return pl.pallas_call(
        body,
        grid=(nv,),
        in_specs=[
            pl.BlockSpec((batch, k), lambda i: (0, 0)),
            pl.BlockSpec((k, _VB), lambda i: (0, i)),
            pl.BlockSpec((batch, 1), lambda i: (0, 0)),
        ],
        out_specs=pl.BlockSpec((batch, _VB), lambda i: (0, i)),
        out_shape=jax.ShapeDtypeStruct((batch, vocab), jnp.float32),
    )(vb, wbt, lse)


def kernel(x, emb, W, b):
    batch, win = x.shape
    vocab, edim = W.shape
    idx = x.reshape(-1)
    v = _gather_mean_sc(idx, emb, batch, win)
    wbt = jnp.concatenate([W.T, b[None, :]], axis=0)
    vb = jnp.concatenate([v, jnp.ones((batch, 1), jnp.float32)], axis=1)
    lse = _lse_tc(vb, wbt, vocab)
    return _logsoftmax_out_tc(vb, wbt, lse, vocab)
